# baseline (speedup 1.0000x reference)
import functools

import jax
import jax.numpy as jnp
from jax import lax
from jax.experimental import pallas as pl
from jax.experimental.pallas import tpu as pltpu
from jax.experimental.pallas import tpu_sc as plsc

N_NODES = 10000
N_EDGES = 320000
D = 128

NC = 2
NS = 16
L = 16
NW = NC * NS

CHUNK = 128
CHUNKS_PW = 80
PAD_EDGES = NW * CHUNKS_PW * CHUNK

HCHUNK = 128
N_HCHUNKS = N_NODES // HCHUNK
HROUNDS = (N_HCHUNKS + NS - 1) // NS
HTAIL = N_NODES - N_HCHUNKS * HCHUNK


def _sc_aggregate(x, meta):
    mesh = plsc.VectorSubcoreMesh(
        core_axis_name="c", subcore_axis_name="s",
        num_cores=NC, num_subcores=NS)

    @functools.partial(
        pl.kernel,
        out_type=jax.ShapeDtypeStruct((NC, N_NODES, D), jnp.float32),
        mesh=mesh,
        scratch_types=[
            pltpu.VMEM_SHARED((N_NODES, D), jnp.float32),
            pltpu.VMEM((CHUNK, D), jnp.float32),
            pltpu.VMEM((CHUNK, D), jnp.float32),
            pltpu.VMEM((3, CHUNK), jnp.int32),
            pltpu.VMEM((3, CHUNK), jnp.int32),
            pltpu.SemaphoreType.DMA,
        ],
    )
    def agg(x_hbm, meta_hbm, out_hbm,
            h_sh, gbuf0, gbuf1, metab0, metab1, sem):
        c = lax.axis_index("c")
        s = lax.axis_index("s")
        wid = c * NS + s
        gbuf = (gbuf0, gbuf1)
        metab = (metab0, metab1)

        def zero_row(r, _):
            for j in range(D // L):
                gbuf0[r, pl.ds(j * L, L)] = jnp.zeros((L,), jnp.float32)
            return 0
        lax.fori_loop(0, HCHUNK, zero_row, 0)
        for k in range(HROUNDS):
            cid = s + NS * k

            @pl.when(cid < N_HCHUNKS)
            def _():
                pltpu.sync_copy(gbuf0, h_sh.at[pl.ds(cid * HCHUNK, HCHUNK)])

        @pl.when(s == NS - 1)
        def _():
            pltpu.sync_copy(gbuf0.at[pl.ds(0, HTAIL)],
                            h_sh.at[pl.ds(N_HCHUNKS * HCHUNK, HTAIL)])
        plsc.subcore_barrier()

        def scale_rows(gb, mb):
            def body(g, _):
                v16 = lax.bitcast_convert_type(
                    mb[2, pl.ds(g * L, L)], jnp.float32)
                for i in range(L):
                    r = g * L + i
                    vs = v16[i]
                    for j in range(D // L):
                        gb[r, pl.ds(j * L, L)] = gb[r, pl.ds(j * L, L)] * vs
                return 0
            lax.fori_loop(0, CHUNK // L, body, 0)

        def fetch(k, b):
            pltpu.sync_copy(meta_hbm.at[wid, k], metab[b])
            pltpu.async_copy(x_hbm.at[metab[b].at[0]], gbuf[b], sem)

        def finish(k, b, prefetch):
            pltpu.make_async_copy(
                x_hbm.at[metab[b].at[0]], gbuf[b], sem).wait()
            if prefetch:
                fetch(k + 1, 1 - b)
            scale_rows(gbuf[b], metab[b])
            pltpu.sync_copy(gbuf[b], h_sh.at[metab[b].at[1]], add=True)

        fetch(0, 0)

        def round2(o, _):
            for b in range(2):
                finish(o * 2 + b, b, prefetch=True)
            return 0
        lax.fori_loop(0, CHUNKS_PW // 2 - 1, round2, 0)
        finish(CHUNKS_PW - 2, 0, prefetch=True)
        finish(CHUNKS_PW - 1, 1, prefetch=False)

        plsc.subcore_barrier()

        for k in range(HROUNDS):
            cid = s + NS * k

            @pl.when(cid < N_HCHUNKS)
            def _():
                pltpu.sync_copy(h_sh.at[pl.ds(cid * HCHUNK, HCHUNK)],
                                out_hbm.at[c, pl.ds(cid * HCHUNK, HCHUNK)])

        @pl.when(s == NS - 1)
        def _():
            pltpu.sync_copy(h_sh.at[pl.ds(N_HCHUNKS * HCHUNK, HTAIL)],
                            out_hbm.at[c, pl.ds(N_HCHUNKS * HCHUNK, HTAIL)])

    return agg(x, meta)


def _tc_matmul_relu(h_partial, W):
    BLOCK = 1000

    def mm(h_ref, w_ref, o_ref):
        hp = h_ref[...]
        y = hp[0] + hp[1]
        o_ref[...] = jnp.maximum(
            jnp.dot(y, w_ref[...], preferred_element_type=jnp.float32), 0.0)

    return pl.pallas_call(
        mm,
        grid=(N_NODES // BLOCK,),
        in_specs=[
            pl.BlockSpec((NC, BLOCK, D), lambda i: (0, i, 0)),
            pl.BlockSpec((D, D), lambda i: (0, 0)),
        ],
        out_specs=pl.BlockSpec((BLOCK, D), lambda i: (i, 0)),
        out_shape=jax.ShapeDtypeStruct((N_NODES, D), jnp.float32),
    )(h_partial, W)


def _pack_meta(adj_indices, adj_values):
    rows = adj_indices[0]
    cols = adj_indices[1]
    vbits = lax.bitcast_convert_type(adj_values, jnp.int32)
    pad = PAD_EDGES - N_EDGES
    zpad = jnp.zeros((pad,), jnp.int32)
    cols_p = jnp.concatenate([cols, zpad])
    rows_p = jnp.concatenate([rows, zpad])
    vbits_p = jnp.concatenate([vbits, zpad])
    meta = jnp.stack([cols_p, rows_p, vbits_p], axis=0)
    meta = meta.reshape(3, NW, CHUNKS_PW, CHUNK).transpose(1, 2, 0, 3)
    return meta


def kernel(input, adj_indices, adj_values, W):
    meta = _pack_meta(adj_indices, adj_values)
    h_partial = _sc_aggregate(input, meta)
    return _tc_matmul_relu(h_partial, W)
